# initial kernel scaffold (unmeasured)
import jax
import jax.numpy as jnp
from jax import lax
from jax.experimental import pallas as pl
from jax.experimental.pallas import tpu as pltpu

N_DEV = 8
N_LOC = 2048
D = 1024
E = 64
E_LOC = E // N_DEV
CAP = 204

_DeviceIdType = getattr(pl, "DeviceIdType", None) or getattr(pltpu, "DeviceIdType")
MESH = _DeviceIdType.MESH
_sem_signal = getattr(pl, "semaphore_signal", None) or getattr(pltpu, "semaphore_signal")
_sem_wait = getattr(pl, "semaphore_wait", None) or getattr(pltpu, "semaphore_wait")
_CompilerParams = getattr(pltpu, "CompilerParams", None) or getattr(
    pltpu, "TPUCompilerParams"
)
_ANY = getattr(pltpu, "ANY", None)
if _ANY is None:
    _ANY = pltpu.MemorySpace.ANY
_VMEM = getattr(pltpu, "VMEM", None)
if _VMEM is None:
    _VMEM = pltpu.MemorySpace.VMEM


def _entry_barrier(my):
    bar = pltpu.get_barrier_semaphore()
    for off in range(1, N_DEV):
        _sem_signal(
            bar,
            inc=1,
            device_id=((my + off) % N_DEV,),
            device_id_type=MESH,
        )
    _sem_wait(bar, N_DEV - 1)


def _allgather_call(x_bf, r2):

    def body(x_ref, r_ref, xall_ref, rall_ref, send_x, recv_x, send_r, recv_r, loc):
        my = lax.axis_index("i")
        _entry_barrier(my)

        cp_x = pltpu.make_async_copy(x_ref, xall_ref.at[my], loc.at[0])
        cp_x.start()
        cp_r = pltpu.make_async_copy(r_ref, rall_ref.at[my], loc.at[1])
        cp_r.start()

        sends = []
        for off in range(1, N_DEV):
            d = (my + off) % N_DEV
            j = off - 1
            sx = pltpu.make_async_remote_copy(
                src_ref=x_ref,
                dst_ref=xall_ref.at[my],
                send_sem=send_x.at[j],
                recv_sem=recv_x.at[j],
                device_id=(d,),
                device_id_type=MESH,
            )
            sx.start()
            sr = pltpu.make_async_remote_copy(
                src_ref=r_ref,
                dst_ref=rall_ref.at[my],
                send_sem=send_r.at[j],
                recv_sem=recv_r.at[j],
                device_id=(d,),
                device_id_type=MESH,
            )
            sr.start()
            sends.append((sx, sr))

        for offp in range(1, N_DEV):
            k = (my - offp) % N_DEV
            j = offp - 1
            rx = pltpu.make_async_remote_copy(
                src_ref=x_ref,
                dst_ref=xall_ref.at[k],
                send_sem=send_x.at[j],
                recv_sem=recv_x.at[j],
                device_id=(k,),
                device_id_type=MESH,
            )
            rx.wait_recv()
            rr = pltpu.make_async_remote_copy(
                src_ref=r_ref,
                dst_ref=rall_ref.at[k],
                send_sem=send_r.at[j],
                recv_sem=recv_r.at[j],
                device_id=(k,),
                device_id_type=MESH,
            )
            rr.wait_recv()

        for sx, sr in sends:
            sx.wait_send()
            sr.wait_send()
        cp_x.wait()
        cp_r.wait()

    return pl.pallas_call(
        body,
        out_shape=(
            jax.ShapeDtypeStruct((N_DEV, N_LOC, D), jnp.bfloat16),
            jax.ShapeDtypeStruct((N_DEV, 16, 128), jnp.int32),
        ),
        in_specs=[
            pl.BlockSpec(memory_space=_VMEM),
            pl.BlockSpec(memory_space=_VMEM),
        ],
        out_specs=(
            pl.BlockSpec(memory_space=_ANY),
            pl.BlockSpec(memory_space=_VMEM),
        ),
        scratch_shapes=[
            pltpu.SemaphoreType.DMA((N_DEV - 1,)),
            pltpu.SemaphoreType.DMA((N_DEV - 1,)),
            pltpu.SemaphoreType.DMA((N_DEV - 1,)),
            pltpu.SemaphoreType.DMA((N_DEV - 1,)),
            pltpu.SemaphoreType.DMA((2,)),
        ],
        compiler_params=_CompilerParams(collective_id=0),
    )(x_bf, r2)


def _combine_call(C):

    def body(c_ref, slots_ref, send_s, recv_s):
        my = lax.axis_index("i")
        _entry_barrier(my)

        sends = []
        for off in range(1, N_DEV):
            d = (my + off) % N_DEV
            j = off - 1
            s = pltpu.make_async_remote_copy(
                src_ref=c_ref.at[d],
                dst_ref=slots_ref.at[j],
                send_sem=send_s.at[j],
                recv_sem=recv_s.at[j],
                device_id=(d,),
                device_id_type=MESH,
            )
            s.start()
            sends.append(s)

        for offp in range(1, N_DEV):
            k = (my - offp) % N_DEV
            j = offp - 1
            r = pltpu.make_async_remote_copy(
                src_ref=c_ref.at[k],
                dst_ref=slots_ref.at[j],
                send_sem=send_s.at[j],
                recv_sem=recv_s.at[j],
                device_id=(k,),
                device_id_type=MESH,
            )
            r.wait_recv()

        for s in sends:
            s.wait_send()

    return pl.pallas_call(
        body,
        out_shape=jax.ShapeDtypeStruct((N_DEV - 1, N_LOC, D), jnp.bfloat16),
        in_specs=[pl.BlockSpec(memory_space=_ANY)],
        out_specs=pl.BlockSpec(memory_space=_ANY),
        scratch_shapes=[
            pltpu.SemaphoreType.DMA((N_DEV - 1,)),
            pltpu.SemaphoreType.DMA((N_DEV - 1,)),
        ],
        compiler_params=_CompilerParams(collective_id=1),
    )(C)


def kernel(x, router_W, route_idx, expert_W):
    del router_W
    my = lax.axis_index("i")
    NT = N_DEV * N_LOC

    x_bf = x.astype(jnp.bfloat16)
    r2 = route_idx.reshape(16, 128)
    X_all, R_all = _allgather_call(x_bf, r2)

    R = R_all.reshape(NT)
    onehot = (R[:, None] == jnp.arange(E, dtype=jnp.int32)[None, :]).astype(jnp.int32)
    inc = jnp.cumsum(onehot, axis=0)
    pos = jnp.take_along_axis(inc - onehot, R[:, None], axis=1)[:, 0]
    accept = pos < CAP

    le = R - my * E_LOC
    local = accept & (le >= 0) & (le < E_LOC)
    flat = jnp.where(local, le * CAP + pos, E_LOC * CAP)
    Gflat = jnp.full((E_LOC * CAP + 1,), NT, dtype=jnp.int32)
    Gflat = Gflat.at[flat].set(jnp.arange(NT, dtype=jnp.int32))
    G = Gflat[: E_LOC * CAP]

    Xg = X_all.reshape(NT, D)[jnp.clip(G, 0, NT - 1)]
    W_bf = expert_W.astype(jnp.bfloat16)
    Y = jnp.einsum(
        "ecd,edf->ecf",
        Xg.reshape(E_LOC, CAP, D),
        W_bf,
        preferred_element_type=jnp.float32,
    )
    C_flat = jnp.zeros((NT + 1, D), dtype=jnp.bfloat16)
    C_flat = C_flat.at[G].set(Y.reshape(E_LOC * CAP, D).astype(jnp.bfloat16))
    C = C_flat[:NT].reshape(N_DEV, N_LOC, D)

    slots = _combine_call(C)
    own = lax.dynamic_index_in_dim(C, my, axis=0, keepdims=False)
    out = own.astype(jnp.float32) + jnp.sum(slots.astype(jnp.float32), axis=0)
    return out


# baseline (device time: 958389 ns/iter reference)
import jax
import jax.numpy as jnp
from jax import lax
from jax.experimental import pallas as pl
from jax.experimental.pallas import tpu as pltpu

N_DEV = 8
N_LOC = 2048
D = 1024
E = 64
E_LOC = E // N_DEV
CAP = 204

_DeviceIdType = getattr(pl, "DeviceIdType", None) or getattr(pltpu, "DeviceIdType")
MESH = _DeviceIdType.MESH
_sem_signal = getattr(pl, "semaphore_signal", None) or getattr(pltpu, "semaphore_signal")
_sem_wait = getattr(pl, "semaphore_wait", None) or getattr(pltpu, "semaphore_wait")
_CompilerParams = getattr(pltpu, "CompilerParams", None) or getattr(
    pltpu, "TPUCompilerParams"
)
_ANY = getattr(pltpu, "ANY", None) or pl.ANY
_VMEM = getattr(pltpu, "VMEM", None) or pltpu.MemorySpace.VMEM


def _entry_barrier(my):
    bar = pltpu.get_barrier_semaphore()
    for off in range(1, N_DEV):
        _sem_signal(
            bar,
            inc=1,
            device_id=((my + off) % N_DEV,),
            device_id_type=MESH,
        )
    _sem_wait(bar, N_DEV - 1)


def _allgather_call(x_bf, r2):

    def body(x_ref, r_ref, xall_ref, rall_ref, send_x, recv_x, send_r, recv_r, loc):
        my = lax.axis_index("i")
        _entry_barrier(my)

        cp_x = pltpu.make_async_copy(x_ref, xall_ref.at[my], loc.at[0])
        cp_x.start()
        cp_r = pltpu.make_async_copy(r_ref, rall_ref.at[my], loc.at[1])
        cp_r.start()

        sends = []
        for off in range(1, N_DEV):
            d = (my + off) % N_DEV
            j = off - 1
            sx = pltpu.make_async_remote_copy(
                src_ref=x_ref,
                dst_ref=xall_ref.at[my],
                send_sem=send_x.at[j],
                recv_sem=recv_x.at[j],
                device_id=(d,),
                device_id_type=MESH,
            )
            sx.start()
            sr = pltpu.make_async_remote_copy(
                src_ref=r_ref,
                dst_ref=rall_ref.at[my],
                send_sem=send_r.at[j],
                recv_sem=recv_r.at[j],
                device_id=(d,),
                device_id_type=MESH,
            )
            sr.start()
            sends.append((sx, sr))

        for offp in range(1, N_DEV):
            k = (my - offp) % N_DEV
            j = offp - 1
            rx = pltpu.make_async_remote_copy(
                src_ref=x_ref,
                dst_ref=xall_ref.at[k],
                send_sem=send_x.at[j],
                recv_sem=recv_x.at[j],
                device_id=(k,),
                device_id_type=MESH,
            )
            rx.wait_recv()
            rr = pltpu.make_async_remote_copy(
                src_ref=r_ref,
                dst_ref=rall_ref.at[k],
                send_sem=send_r.at[j],
                recv_sem=recv_r.at[j],
                device_id=(k,),
                device_id_type=MESH,
            )
            rr.wait_recv()

        for sx, sr in sends:
            sx.wait_send()
            sr.wait_send()
        cp_x.wait()
        cp_r.wait()

    return pl.pallas_call(
        body,
        out_shape=(
            jax.ShapeDtypeStruct((N_DEV, N_LOC, D), jnp.bfloat16),
            jax.ShapeDtypeStruct((N_DEV, 16, 128), jnp.int32),
        ),
        in_specs=[
            pl.BlockSpec(memory_space=_VMEM),
            pl.BlockSpec(memory_space=_VMEM),
        ],
        out_specs=(
            pl.BlockSpec(memory_space=_ANY),
            pl.BlockSpec(memory_space=_VMEM),
        ),
        scratch_shapes=[
            pltpu.SemaphoreType.DMA((N_DEV - 1,)),
            pltpu.SemaphoreType.DMA((N_DEV - 1,)),
            pltpu.SemaphoreType.DMA((N_DEV - 1,)),
            pltpu.SemaphoreType.DMA((N_DEV - 1,)),
            pltpu.SemaphoreType.DMA((2,)),
        ],
        compiler_params=_CompilerParams(collective_id=0),
    )(x_bf, r2)


def _combine_call(C):

    def body(c_ref, slots_ref, send_s, recv_s):
        my = lax.axis_index("i")
        _entry_barrier(my)

        sends = []
        for off in range(1, N_DEV):
            d = (my + off) % N_DEV
            j = off - 1
            s = pltpu.make_async_remote_copy(
                src_ref=c_ref.at[d],
                dst_ref=slots_ref.at[j],
                send_sem=send_s.at[j],
                recv_sem=recv_s.at[j],
                device_id=(d,),
                device_id_type=MESH,
            )
            s.start()
            sends.append(s)

        for offp in range(1, N_DEV):
            k = (my - offp) % N_DEV
            j = offp - 1
            r = pltpu.make_async_remote_copy(
                src_ref=c_ref.at[k],
                dst_ref=slots_ref.at[j],
                send_sem=send_s.at[j],
                recv_sem=recv_s.at[j],
                device_id=(k,),
                device_id_type=MESH,
            )
            r.wait_recv()

        for s in sends:
            s.wait_send()

    return pl.pallas_call(
        body,
        out_shape=jax.ShapeDtypeStruct((N_DEV - 1, N_LOC, D), jnp.bfloat16),
        in_specs=[pl.BlockSpec(memory_space=_ANY)],
        out_specs=pl.BlockSpec(memory_space=_ANY),
        scratch_shapes=[
            pltpu.SemaphoreType.DMA((N_DEV - 1,)),
            pltpu.SemaphoreType.DMA((N_DEV - 1,)),
        ],
        compiler_params=_CompilerParams(collective_id=1),
    )(C)


def kernel(x, router_W, route_idx, expert_W):
    del router_W
    my = lax.axis_index("i")
    NT = N_DEV * N_LOC

    x_bf = x.astype(jnp.bfloat16)
    r2 = route_idx.reshape(16, 128)
    X_all, R_all = _allgather_call(x_bf, r2)

    R = R_all.reshape(NT)
    onehot = (R[:, None] == jnp.arange(E, dtype=jnp.int32)[None, :]).astype(jnp.int32)
    inc = jnp.cumsum(onehot, axis=0)
    pos = jnp.take_along_axis(inc - onehot, R[:, None], axis=1)[:, 0]
    accept = pos < CAP

    le = R - my * E_LOC
    local = accept & (le >= 0) & (le < E_LOC)
    flat = jnp.where(local, le * CAP + pos, E_LOC * CAP)
    Gflat = jnp.full((E_LOC * CAP + 1,), NT, dtype=jnp.int32)
    Gflat = Gflat.at[flat].set(jnp.arange(NT, dtype=jnp.int32))
    G = Gflat[: E_LOC * CAP]

    Xg = X_all.reshape(NT, D)[jnp.clip(G, 0, NT - 1)]
    W_bf = expert_W.astype(jnp.bfloat16)
    Y = jnp.einsum(
        "ecd,edf->ecf",
        Xg.reshape(E_LOC, CAP, D),
        W_bf,
        preferred_element_type=jnp.float32,
    )
    C_flat = jnp.zeros((NT + 1, D), dtype=jnp.bfloat16)
    C_flat = C_flat.at[G].set(Y.reshape(E_LOC * CAP, D).astype(jnp.bfloat16))
    C = C_flat[:NT].reshape(N_DEV, N_LOC, D)

    slots = _combine_call(C)
    own = lax.dynamic_index_in_dim(C, my, axis=0, keepdims=False)
    out = own.astype(jnp.float32) + jnp.sum(slots.astype(jnp.float32), axis=0)
    return out


# device time: 279473 ns/iter; 3.4293x vs baseline; 3.4293x over previous
import jax
import jax.numpy as jnp
from jax import lax
from jax.experimental import pallas as pl
from jax.experimental.pallas import tpu as pltpu

N_DEV = 8
N_LOC = 2048
D = 1024
E = 64
E_LOC = E // N_DEV
CAP = 204
PAD = 320
ECAP = 256
NP = N_DEV * PAD
NT = N_DEV * N_LOC
BIG = jnp.int32(1_000_000)

_DeviceIdType = getattr(pl, "DeviceIdType", None) or getattr(pltpu, "DeviceIdType")
MESH = _DeviceIdType.MESH
_sem_signal = getattr(pl, "semaphore_signal", None) or getattr(pltpu, "semaphore_signal")
_sem_wait = getattr(pl, "semaphore_wait", None) or getattr(pltpu, "semaphore_wait")
_CompilerParams = getattr(pltpu, "CompilerParams", None) or getattr(
    pltpu, "TPUCompilerParams"
)
_ANY = getattr(pltpu, "ANY", None) or pl.ANY
_VMEM = getattr(pltpu, "VMEM", None) or pltpu.MemorySpace.VMEM


def _entry_barrier(my):
    bar = pltpu.get_barrier_semaphore()
    for off in range(1, N_DEV):
        _sem_signal(
            bar,
            inc=1,
            device_id=((my + off) % N_DEV,),
            device_id_type=MESH,
        )
    _sem_wait(bar, N_DEV - 1)


def _a2a_sections(my, src_ref, dst_ref, send_sems, recv_sems):
    sends = []
    for off in range(1, N_DEV):
        d = (my + off) % N_DEV
        j = off - 1
        s = pltpu.make_async_remote_copy(
            src_ref=src_ref.at[pl.ds(d * PAD, PAD), :],
            dst_ref=dst_ref.at[pl.ds(my * PAD, PAD), :],
            send_sem=send_sems.at[j],
            recv_sem=recv_sems.at[j],
            device_id=(d,),
            device_id_type=MESH,
        )
        s.start()
        sends.append(s)
    for offp in range(1, N_DEV):
        k = (my - offp) % N_DEV
        j = offp - 1
        r = pltpu.make_async_remote_copy(
            src_ref=src_ref.at[pl.ds(k * PAD, PAD), :],
            dst_ref=dst_ref.at[pl.ds(k * PAD, PAD), :],
            send_sem=send_sems.at[j],
            recv_sem=recv_sems.at[j],
            device_id=(k,),
            device_id_type=MESH,
        )
        r.wait_recv()
    for s in sends:
        s.wait_send()


def _route_allgather(r2):

    def body(r_ref, rall_ref, send_s, recv_s, loc):
        my = lax.axis_index("i")
        _entry_barrier(my)
        cp = pltpu.make_async_copy(r_ref, rall_ref.at[my], loc)
        cp.start()
        sends = []
        for off in range(1, N_DEV):
            d = (my + off) % N_DEV
            j = off - 1
            s = pltpu.make_async_remote_copy(
                src_ref=r_ref,
                dst_ref=rall_ref.at[my],
                send_sem=send_s.at[j],
                recv_sem=recv_s.at[j],
                device_id=(d,),
                device_id_type=MESH,
            )
            s.start()
            sends.append(s)
        for offp in range(1, N_DEV):
            k = (my - offp) % N_DEV
            j = offp - 1
            r = pltpu.make_async_remote_copy(
                src_ref=r_ref,
                dst_ref=rall_ref.at[k],
                send_sem=send_s.at[j],
                recv_sem=recv_s.at[j],
                device_id=(k,),
                device_id_type=MESH,
            )
            r.wait_recv()
        for s in sends:
            s.wait_send()
        cp.wait()

    return pl.pallas_call(
        body,
        out_shape=jax.ShapeDtypeStruct((N_DEV, 16, 128), jnp.int32),
        in_specs=[pl.BlockSpec(memory_space=_VMEM)],
        out_specs=pl.BlockSpec(memory_space=_VMEM),
        scratch_shapes=[
            pltpu.SemaphoreType.DMA((N_DEV - 1,)),
            pltpu.SemaphoreType.DMA((N_DEV - 1,)),
            pltpu.SemaphoreType.DMA,
        ],
        compiler_params=_CompilerParams(collective_id=0),
    )(r2)


def _pack_dispatch(P, x_bf):

    def body(p_ref, x_ref, recv_ref, packed_ref, send_s, recv_s, loc):
        my = lax.axis_index("i")
        _entry_barrier(my)
        packed = jnp.dot(
            p_ref[...], x_ref[...], preferred_element_type=jnp.float32
        )
        packed_ref[...] = packed.astype(jnp.bfloat16)
        cp = pltpu.make_async_copy(
            packed_ref.at[pl.ds(my * PAD, PAD), :],
            recv_ref.at[pl.ds(my * PAD, PAD), :],
            loc,
        )
        cp.start()
        _a2a_sections(my, packed_ref, recv_ref, send_s, recv_s)
        cp.wait()

    return pl.pallas_call(
        body,
        out_shape=jax.ShapeDtypeStruct((NP, D), jnp.bfloat16),
        in_specs=[
            pl.BlockSpec(memory_space=_VMEM),
            pl.BlockSpec(memory_space=_VMEM),
        ],
        out_specs=pl.BlockSpec(memory_space=_VMEM),
        scratch_shapes=[
            pltpu.VMEM((NP, D), jnp.bfloat16),
            pltpu.SemaphoreType.DMA((N_DEV - 1,)),
            pltpu.SemaphoreType.DMA((N_DEV - 1,)),
            pltpu.SemaphoreType.DMA,
        ],
        compiler_params=_CompilerParams(collective_id=1),
    )(P, x_bf)


def _expert_compute(Eoh, recv, W_bf):

    def body(e_ref, r_ref, w_ref, y_ref, wbuf, wsem):
        xg = jnp.dot(
            e_ref[...], r_ref[...], preferred_element_type=jnp.float32
        ).astype(jnp.bfloat16)
        for e in range(E_LOC):
            cp = pltpu.make_async_copy(w_ref.at[e], wbuf, wsem)
            cp.start()
            cp.wait()
            y = jnp.dot(
                xg[e * ECAP : (e + 1) * ECAP, :],
                wbuf[...],
                preferred_element_type=jnp.float32,
            )
            y_ref[e * ECAP : (e + 1) * ECAP, :] = y.astype(jnp.bfloat16)

    return pl.pallas_call(
        body,
        out_shape=jax.ShapeDtypeStruct((E_LOC * ECAP, D), jnp.bfloat16),
        in_specs=[
            pl.BlockSpec(memory_space=_VMEM),
            pl.BlockSpec(memory_space=_VMEM),
            pl.BlockSpec(memory_space=_ANY),
        ],
        out_specs=pl.BlockSpec(memory_space=_VMEM),
        scratch_shapes=[
            pltpu.VMEM((D, D), jnp.bfloat16),
            pltpu.SemaphoreType.DMA,
        ],
    )(Eoh, recv, W_bf)


def _scatter_return(Et, Y):

    def body(et_ref, y_ref, ret_ref, yp_ref, send_s, recv_s, loc):
        my = lax.axis_index("i")
        _entry_barrier(my)
        yp = jnp.dot(
            et_ref[...], y_ref[...], preferred_element_type=jnp.float32
        )
        yp_ref[...] = yp.astype(jnp.bfloat16)
        cp = pltpu.make_async_copy(
            yp_ref.at[pl.ds(my * PAD, PAD), :],
            ret_ref.at[pl.ds(my * PAD, PAD), :],
            loc,
        )
        cp.start()
        _a2a_sections(my, yp_ref, ret_ref, send_s, recv_s)
        cp.wait()

    return pl.pallas_call(
        body,
        out_shape=jax.ShapeDtypeStruct((NP, D), jnp.bfloat16),
        in_specs=[
            pl.BlockSpec(memory_space=_VMEM),
            pl.BlockSpec(memory_space=_VMEM),
        ],
        out_specs=pl.BlockSpec(memory_space=_VMEM),
        scratch_shapes=[
            pltpu.VMEM((NP, D), jnp.bfloat16),
            pltpu.SemaphoreType.DMA((N_DEV - 1,)),
            pltpu.SemaphoreType.DMA((N_DEV - 1,)),
            pltpu.SemaphoreType.DMA,
        ],
        compiler_params=_CompilerParams(collective_id=2),
    )(Et, Y)


def _unpack(Pt, ret):

    def body(pt_ref, ret_ref, o_ref):
        o_ref[...] = jnp.dot(
            pt_ref[...], ret_ref[...], preferred_element_type=jnp.float32
        )

    return pl.pallas_call(
        body,
        out_shape=jax.ShapeDtypeStruct((N_LOC, D), jnp.float32),
        in_specs=[
            pl.BlockSpec(memory_space=_VMEM),
            pl.BlockSpec(memory_space=_VMEM),
        ],
        out_specs=pl.BlockSpec(memory_space=_VMEM),
    )(Pt, ret)


def kernel(x, router_W, route_idx, expert_W):
    del router_W
    my = lax.axis_index("i")

    R_all = _route_allgather(route_idx.reshape(16, 128))
    R = R_all.reshape(NT)

    onehot = (R[:, None] == jnp.arange(E, dtype=jnp.int32)[None, :]).astype(
        jnp.int32
    )
    inc = jnp.cumsum(onehot, axis=0)
    pos = jnp.sum(inc * onehot, axis=1) - 1
    accept = pos < CAP
    dest = R // E_LOC

    d_loc = lax.dynamic_slice_in_dim(dest, my * N_LOC, N_LOC)
    a_loc = lax.dynamic_slice_in_dim(accept.astype(jnp.int32), my * N_LOC, N_LOC)
    oh8 = (d_loc[:, None] == jnp.arange(N_DEV, dtype=jnp.int32)[None, :]).astype(
        jnp.int32
    ) * a_loc[:, None]
    slot_loc = jnp.sum((jnp.cumsum(oh8, axis=0) - oh8) * oh8, axis=1)
    valid_loc = (a_loc == 1) & (slot_loc < PAD)
    key_loc = jnp.where(valid_loc, d_loc * PAD + slot_loc, BIG)
    rows = jnp.arange(NP, dtype=jnp.int32)
    P = (rows[:, None] == key_loc[None, :]).astype(jnp.bfloat16)
    Pt = (key_loc[:, None] == rows[None, :]).astype(jnp.bfloat16)

    mask_my = (accept & (dest == my)).astype(jnp.int32)
    seg = mask_my.reshape(N_DEV, N_LOC)
    slot_g = (jnp.cumsum(seg, axis=1) - seg).reshape(NT)
    valid_g = (mask_my == 1) & (slot_g < PAD)
    crow = jnp.where(
        valid_g, (jnp.arange(NT, dtype=jnp.int32) // N_LOC) * PAD + slot_g, NP
    )
    erow = (R - my * E_LOC) * ECAP + pos
    col_key = jnp.full((NP + 1,), BIG, dtype=jnp.int32).at[crow].set(erow)
    col_key = col_key[:NP]
    erows = jnp.arange(E_LOC * ECAP, dtype=jnp.int32)
    Eoh = (erows[:, None] == col_key[None, :]).astype(jnp.bfloat16)
    Et = (col_key[:, None] == erows[None, :]).astype(jnp.bfloat16)

    x_bf = x.astype(jnp.bfloat16)
    W_bf = expert_W.astype(jnp.bfloat16)

    recv = _pack_dispatch(P, x_bf)
    Y = _expert_compute(Eoh, recv, W_bf)
    ret = _scatter_return(Et, Y)
    out = _unpack(Pt, ret)
    return out


# device time: 196122 ns/iter; 4.8867x vs baseline; 1.4250x over previous
import jax
import jax.numpy as jnp
from jax import lax
from jax.experimental import pallas as pl
from jax.experimental.pallas import tpu as pltpu

N_DEV = 8
N_LOC = 2048
D = 1024
DA = D + 128
E = 64
E_LOC = E // N_DEV
CAP = 204
PAD = 320
ECAP = 256
NP = N_DEV * PAD
NT = N_DEV * N_LOC
BIG = jnp.int32(1_000_000)

_DeviceIdType = getattr(pl, "DeviceIdType", None) or getattr(pltpu, "DeviceIdType")
MESH = _DeviceIdType.MESH
_sem_signal = getattr(pl, "semaphore_signal", None) or getattr(pltpu, "semaphore_signal")
_sem_wait = getattr(pl, "semaphore_wait", None) or getattr(pltpu, "semaphore_wait")
_CompilerParams = getattr(pltpu, "CompilerParams", None) or getattr(
    pltpu, "TPUCompilerParams"
)
_ANY = getattr(pltpu, "ANY", None) or pl.ANY
_VMEM = getattr(pltpu, "VMEM", None) or pltpu.MemorySpace.VMEM


def _entry_barrier(my):
    bar = pltpu.get_barrier_semaphore()
    for off in range(1, N_DEV):
        _sem_signal(
            bar,
            inc=1,
            device_id=((my + off) % N_DEV,),
            device_id_type=MESH,
        )
    _sem_wait(bar, N_DEV - 1)


def _a2a_sections(my, src_ref, dst_ref, send_sems, recv_sems):
    sends = []
    for off in range(1, N_DEV):
        d = (my + off) % N_DEV
        j = off - 1
        s = pltpu.make_async_remote_copy(
            src_ref=src_ref.at[pl.ds(d * PAD, PAD), :],
            dst_ref=dst_ref.at[pl.ds(my * PAD, PAD), :],
            send_sem=send_sems.at[j],
            recv_sem=recv_sems.at[j],
            device_id=(d,),
            device_id_type=MESH,
        )
        s.start()
        sends.append(s)
    for offp in range(1, N_DEV):
        k = (my - offp) % N_DEV
        j = offp - 1
        r = pltpu.make_async_remote_copy(
            src_ref=src_ref.at[pl.ds(k * PAD, PAD), :],
            dst_ref=dst_ref.at[pl.ds(k * PAD, PAD), :],
            send_sem=send_sems.at[j],
            recv_sem=recv_sems.at[j],
            device_id=(k,),
            device_id_type=MESH,
        )
        r.wait_recv()
    for s in sends:
        s.wait_send()


def _route_allgather(r2):

    def body(r_ref, rall_ref, send_s, recv_s, loc):
        my = lax.axis_index("i")
        _entry_barrier(my)
        cp = pltpu.make_async_copy(r_ref, rall_ref.at[my], loc)
        cp.start()
        sends = []
        for off in range(1, N_DEV):
            d = (my + off) % N_DEV
            j = off - 1
            s = pltpu.make_async_remote_copy(
                src_ref=r_ref,
                dst_ref=rall_ref.at[my],
                send_sem=send_s.at[j],
                recv_sem=recv_s.at[j],
                device_id=(d,),
                device_id_type=MESH,
            )
            s.start()
            sends.append(s)
        for offp in range(1, N_DEV):
            k = (my - offp) % N_DEV
            j = offp - 1
            r = pltpu.make_async_remote_copy(
                src_ref=r_ref,
                dst_ref=rall_ref.at[k],
                send_sem=send_s.at[j],
                recv_sem=recv_s.at[j],
                device_id=(k,),
                device_id_type=MESH,
            )
            r.wait_recv()
        for s in sends:
            s.wait_send()
        cp.wait()

    return pl.pallas_call(
        body,
        out_shape=jax.ShapeDtypeStruct((N_DEV, 16, 128), jnp.int32),
        in_specs=[pl.BlockSpec(memory_space=_VMEM)],
        out_specs=pl.BlockSpec(memory_space=_VMEM),
        scratch_shapes=[
            pltpu.SemaphoreType.DMA((N_DEV - 1,)),
            pltpu.SemaphoreType.DMA((N_DEV - 1,)),
            pltpu.SemaphoreType.DMA,
        ],
        compiler_params=_CompilerParams(collective_id=0),
    )(r2)


def _pack_dispatch(P, x_aug):

    def body(p_ref, x_ref, recv_ref, packed_ref, send_s, recv_s, loc):
        my = lax.axis_index("i")
        _entry_barrier(my)
        packed = jnp.dot(
            p_ref[...], x_ref[...], preferred_element_type=jnp.float32
        )
        packed_ref[...] = packed.astype(jnp.bfloat16)
        cp = pltpu.make_async_copy(
            packed_ref.at[pl.ds(my * PAD, PAD), :],
            recv_ref.at[pl.ds(my * PAD, PAD), :],
            loc,
        )
        cp.start()
        _a2a_sections(my, packed_ref, recv_ref, send_s, recv_s)
        cp.wait()

    return pl.pallas_call(
        body,
        out_shape=jax.ShapeDtypeStruct((NP, DA), jnp.bfloat16),
        in_specs=[
            pl.BlockSpec(memory_space=_VMEM),
            pl.BlockSpec(memory_space=_VMEM),
        ],
        out_specs=pl.BlockSpec(memory_space=_VMEM),
        scratch_shapes=[
            pltpu.VMEM((NP, DA), jnp.bfloat16),
            pltpu.SemaphoreType.DMA((N_DEV - 1,)),
            pltpu.SemaphoreType.DMA((N_DEV - 1,)),
            pltpu.SemaphoreType.DMA,
        ],
        compiler_params=_CompilerParams(collective_id=1),
    )(P, x_aug)


def _expert_compute(Eoh, recv, W):

    def body(e_ref, r_ref, w_ref, y_ref, wbuf, wsems):
        cp0 = pltpu.make_async_copy(w_ref.at[0], wbuf.at[0], wsems.at[0])
        cp0.start()
        xg = jnp.dot(
            e_ref[...], r_ref[:, :D], preferred_element_type=jnp.float32
        ).astype(jnp.bfloat16)
        for e in range(E_LOC):
            pltpu.make_async_copy(
                w_ref.at[e], wbuf.at[e % 2], wsems.at[e % 2]
            ).wait()
            if e + 1 < E_LOC:
                pltpu.make_async_copy(
                    w_ref.at[e + 1], wbuf.at[(e + 1) % 2], wsems.at[(e + 1) % 2]
                ).start()
            y = jnp.dot(
                xg[e * ECAP : (e + 1) * ECAP, :],
                wbuf[e % 2].astype(jnp.bfloat16),
                preferred_element_type=jnp.float32,
            )
            y_ref[e * ECAP : (e + 1) * ECAP, :] = y.astype(jnp.bfloat16)

    return pl.pallas_call(
        body,
        out_shape=jax.ShapeDtypeStruct((E_LOC * ECAP, D), jnp.bfloat16),
        in_specs=[
            pl.BlockSpec(memory_space=_VMEM),
            pl.BlockSpec(memory_space=_VMEM),
            pl.BlockSpec(memory_space=_ANY),
        ],
        out_specs=pl.BlockSpec(memory_space=_VMEM),
        scratch_shapes=[
            pltpu.VMEM((2, D, D), jnp.float32),
            pltpu.SemaphoreType.DMA((2,)),
        ],
    )(Eoh, recv, W)


def _scatter_return(Et, Y):

    def body(et_ref, y_ref, ret_ref, yp_ref, send_s, recv_s, loc):
        my = lax.axis_index("i")
        _entry_barrier(my)
        yp = jnp.dot(
            et_ref[...], y_ref[...], preferred_element_type=jnp.float32
        )
        yp_ref[...] = yp.astype(jnp.bfloat16)
        cp = pltpu.make_async_copy(
            yp_ref.at[pl.ds(my * PAD, PAD), :],
            ret_ref.at[pl.ds(my * PAD, PAD), :],
            loc,
        )
        cp.start()
        _a2a_sections(my, yp_ref, ret_ref, send_s, recv_s)
        cp.wait()

    return pl.pallas_call(
        body,
        out_shape=jax.ShapeDtypeStruct((NP, D), jnp.bfloat16),
        in_specs=[
            pl.BlockSpec(memory_space=_VMEM),
            pl.BlockSpec(memory_space=_VMEM),
        ],
        out_specs=pl.BlockSpec(memory_space=_VMEM),
        scratch_shapes=[
            pltpu.VMEM((NP, D), jnp.bfloat16),
            pltpu.SemaphoreType.DMA((N_DEV - 1,)),
            pltpu.SemaphoreType.DMA((N_DEV - 1,)),
            pltpu.SemaphoreType.DMA,
        ],
        compiler_params=_CompilerParams(collective_id=2),
    )(Et, Y)


def _unpack(Pt, ret):

    def body(pt_ref, ret_ref, o_ref):
        o_ref[...] = jnp.dot(
            pt_ref[...], ret_ref[...], preferred_element_type=jnp.float32
        )

    return pl.pallas_call(
        body,
        out_shape=jax.ShapeDtypeStruct((N_LOC, D), jnp.float32),
        in_specs=[
            pl.BlockSpec(memory_space=_VMEM),
            pl.BlockSpec(memory_space=_VMEM),
        ],
        out_specs=pl.BlockSpec(memory_space=_VMEM),
    )(Pt, ret)


def kernel(x, router_W, route_idx, expert_W):
    del router_W
    my = lax.axis_index("i")

    R_all = _route_allgather(route_idx.reshape(16, 128))
    R = R_all.reshape(NT)

    onehot = (R[:, None] == jnp.arange(E, dtype=jnp.int32)[None, :]).astype(
        jnp.int32
    )
    inc = jnp.cumsum(onehot, axis=0)
    pos = jnp.sum(inc * onehot, axis=1) - 1
    accept = pos < CAP
    dest = R // E_LOC

    d_loc = lax.dynamic_slice_in_dim(dest, my * N_LOC, N_LOC)
    a_loc = lax.dynamic_slice_in_dim(accept.astype(jnp.int32), my * N_LOC, N_LOC)
    p_loc = lax.dynamic_slice_in_dim(pos, my * N_LOC, N_LOC)
    le_loc = lax.dynamic_slice_in_dim(R % E_LOC, my * N_LOC, N_LOC)
    oh8 = (d_loc[:, None] == jnp.arange(N_DEV, dtype=jnp.int32)[None, :]).astype(
        jnp.int32
    ) * a_loc[:, None]
    slot_loc = jnp.sum((jnp.cumsum(oh8, axis=0) - oh8) * oh8, axis=1)
    valid_loc = (a_loc == 1) & (slot_loc < PAD)
    key_loc = jnp.where(valid_loc, d_loc * PAD + slot_loc, BIG)
    rows = jnp.arange(NP, dtype=jnp.int32)
    P = (rows[:, None] == key_loc[None, :]).astype(jnp.bfloat16)
    Pt = (key_loc[:, None] == rows[None, :]).astype(jnp.bfloat16)

    lane = jnp.arange(128, dtype=jnp.int32)[None, :]
    augf = jnp.where(
        lane == 0,
        le_loc[:, None].astype(jnp.float32),
        jnp.where(
            lane == 1,
            p_loc[:, None].astype(jnp.float32),
            jnp.where(lane == 2, 1.0, 0.0),
        ),
    )
    x_aug = jnp.concatenate(
        [x.astype(jnp.bfloat16), augf.astype(jnp.bfloat16)], axis=1
    )

    recv = _pack_dispatch(P, x_aug)

    le_r = recv[:, D].astype(jnp.int32)
    pos_r = recv[:, D + 1].astype(jnp.int32)
    valid_r = recv[:, D + 2] > 0.5
    col_key = jnp.where(valid_r, le_r * ECAP + pos_r, BIG)
    erows = jnp.arange(E_LOC * ECAP, dtype=jnp.int32)
    Eoh = (erows[:, None] == col_key[None, :]).astype(jnp.bfloat16)
    Et = (col_key[:, None] == erows[None, :]).astype(jnp.bfloat16)

    Y = _expert_compute(Eoh, recv, expert_W)
    ret = _scatter_return(Et, Y)
    out = _unpack(Pt, ret)
    return out


# device time: 178809 ns/iter; 5.3598x vs baseline; 1.0968x over previous
import jax
import jax.numpy as jnp
from jax import lax
from jax.experimental import pallas as pl
from jax.experimental.pallas import tpu as pltpu

N_DEV = 8
N_LOC = 2048
D = 1024
DA = D + 128
E = 64
E_LOC = E // N_DEV
CAP = 204
PAD = 320
ECAP = 256
NP = N_DEV * PAD
NT = N_DEV * N_LOC
BIG = jnp.int32(1_000_000)

_DeviceIdType = getattr(pl, "DeviceIdType", None) or getattr(pltpu, "DeviceIdType")
MESH = _DeviceIdType.MESH
_sem_signal = getattr(pl, "semaphore_signal", None) or getattr(pltpu, "semaphore_signal")
_sem_wait = getattr(pl, "semaphore_wait", None) or getattr(pltpu, "semaphore_wait")
_CompilerParams = getattr(pltpu, "CompilerParams", None) or getattr(
    pltpu, "TPUCompilerParams"
)
_ANY = getattr(pltpu, "ANY", None) or pl.ANY
_VMEM = getattr(pltpu, "VMEM", None) or pltpu.MemorySpace.VMEM


def _entry_barrier(my):
    bar = pltpu.get_barrier_semaphore()
    for off in range(1, N_DEV):
        _sem_signal(
            bar,
            inc=1,
            device_id=((my + off) % N_DEV,),
            device_id_type=MESH,
        )
    _sem_wait(bar, N_DEV - 1)


def _a2a_sections(my, src_ref, dst_ref, send_sems, recv_sems):
    sends = []
    for off in range(1, N_DEV):
        d = (my + off) % N_DEV
        j = off - 1
        s = pltpu.make_async_remote_copy(
            src_ref=src_ref.at[pl.ds(d * PAD, PAD), :],
            dst_ref=dst_ref.at[pl.ds(my * PAD, PAD), :],
            send_sem=send_sems.at[j],
            recv_sem=recv_sems.at[j],
            device_id=(d,),
            device_id_type=MESH,
        )
        s.start()
        sends.append(s)
    for offp in range(1, N_DEV):
        k = (my - offp) % N_DEV
        j = offp - 1
        r = pltpu.make_async_remote_copy(
            src_ref=src_ref.at[pl.ds(k * PAD, PAD), :],
            dst_ref=dst_ref.at[pl.ds(k * PAD, PAD), :],
            send_sem=send_sems.at[j],
            recv_sem=recv_sems.at[j],
            device_id=(k,),
            device_id_type=MESH,
        )
        r.wait_recv()
    for s in sends:
        s.wait_send()


def _route_allgather(r2):

    def body(r_ref, rall_ref, send_s, recv_s, loc):
        my = lax.axis_index("i")
        _entry_barrier(my)
        cp = pltpu.make_async_copy(r_ref, rall_ref.at[my], loc)
        cp.start()
        sends = []
        for off in range(1, N_DEV):
            d = (my + off) % N_DEV
            j = off - 1
            s = pltpu.make_async_remote_copy(
                src_ref=r_ref,
                dst_ref=rall_ref.at[my],
                send_sem=send_s.at[j],
                recv_sem=recv_s.at[j],
                device_id=(d,),
                device_id_type=MESH,
            )
            s.start()
            sends.append(s)
        for offp in range(1, N_DEV):
            k = (my - offp) % N_DEV
            j = offp - 1
            r = pltpu.make_async_remote_copy(
                src_ref=r_ref,
                dst_ref=rall_ref.at[k],
                send_sem=send_s.at[j],
                recv_sem=recv_s.at[j],
                device_id=(k,),
                device_id_type=MESH,
            )
            r.wait_recv()
        for s in sends:
            s.wait_send()
        cp.wait()

    return pl.pallas_call(
        body,
        out_shape=jax.ShapeDtypeStruct((N_DEV, 16, 128), jnp.int32),
        in_specs=[pl.BlockSpec(memory_space=_VMEM)],
        out_specs=pl.BlockSpec(memory_space=_VMEM),
        scratch_shapes=[
            pltpu.SemaphoreType.DMA((N_DEV - 1,)),
            pltpu.SemaphoreType.DMA((N_DEV - 1,)),
            pltpu.SemaphoreType.DMA,
        ],
        compiler_params=_CompilerParams(collective_id=0),
    )(r2)


def _pipelined_pack_a2a(my, m_ref, rhs_ref, dst_ref, stage_ref, send_s, recv_s, loc):
    sends = []
    cp = None
    for off in range(N_DEV):
        d = (my + off) % N_DEV
        sec = jnp.dot(
            m_ref[pl.ds(d * PAD, PAD), :],
            rhs_ref[...],
            preferred_element_type=jnp.float32,
        ).astype(stage_ref.dtype)
        stage_ref[pl.ds(d * PAD, PAD), :] = sec
        if off == 0:
            cp = pltpu.make_async_copy(
                stage_ref.at[pl.ds(my * PAD, PAD), :],
                dst_ref.at[pl.ds(my * PAD, PAD), :],
                loc,
            )
            cp.start()
        else:
            j = off - 1
            s = pltpu.make_async_remote_copy(
                src_ref=stage_ref.at[pl.ds(d * PAD, PAD), :],
                dst_ref=dst_ref.at[pl.ds(my * PAD, PAD), :],
                send_sem=send_s.at[j],
                recv_sem=recv_s.at[j],
                device_id=(d,),
                device_id_type=MESH,
            )
            s.start()
            sends.append(s)
    for offp in range(1, N_DEV):
        k = (my - offp) % N_DEV
        j = offp - 1
        r = pltpu.make_async_remote_copy(
            src_ref=stage_ref.at[pl.ds(k * PAD, PAD), :],
            dst_ref=dst_ref.at[pl.ds(k * PAD, PAD), :],
            send_sem=send_s.at[j],
            recv_sem=recv_s.at[j],
            device_id=(k,),
            device_id_type=MESH,
        )
        r.wait_recv()
    for s in sends:
        s.wait_send()
    cp.wait()


def _pack_dispatch(P, x_aug):

    def body(p_ref, x_ref, recv_ref, packed_ref, send_s, recv_s, loc):
        my = lax.axis_index("i")
        _entry_barrier(my)
        _pipelined_pack_a2a(
            my, p_ref, x_ref, recv_ref, packed_ref, send_s, recv_s, loc
        )

    return pl.pallas_call(
        body,
        out_shape=jax.ShapeDtypeStruct((NP, DA), jnp.bfloat16),
        in_specs=[
            pl.BlockSpec(memory_space=_VMEM),
            pl.BlockSpec(memory_space=_VMEM),
        ],
        out_specs=pl.BlockSpec(memory_space=_VMEM),
        scratch_shapes=[
            pltpu.VMEM((NP, DA), jnp.bfloat16),
            pltpu.SemaphoreType.DMA((N_DEV - 1,)),
            pltpu.SemaphoreType.DMA((N_DEV - 1,)),
            pltpu.SemaphoreType.DMA,
        ],
        compiler_params=_CompilerParams(collective_id=1),
    )(P, x_aug)


def _expert_compute(Eoh, recv, W):

    def body(e_ref, r_ref, w_ref, y_ref, wbuf, wsems):
        cp0 = pltpu.make_async_copy(w_ref.at[0], wbuf.at[0], wsems.at[0])
        cp0.start()
        xg = jnp.dot(
            e_ref[...], r_ref[:, :D], preferred_element_type=jnp.float32
        ).astype(jnp.bfloat16)
        for e in range(E_LOC):
            pltpu.make_async_copy(
                w_ref.at[e], wbuf.at[e % 2], wsems.at[e % 2]
            ).wait()
            if e + 1 < E_LOC:
                pltpu.make_async_copy(
                    w_ref.at[e + 1], wbuf.at[(e + 1) % 2], wsems.at[(e + 1) % 2]
                ).start()
            y = jnp.dot(
                xg[e * ECAP : (e + 1) * ECAP, :],
                wbuf[e % 2].astype(jnp.bfloat16),
                preferred_element_type=jnp.float32,
            )
            y_ref[e * ECAP : (e + 1) * ECAP, :] = y.astype(jnp.bfloat16)

    return pl.pallas_call(
        body,
        out_shape=jax.ShapeDtypeStruct((E_LOC * ECAP, D), jnp.bfloat16),
        in_specs=[
            pl.BlockSpec(memory_space=_VMEM),
            pl.BlockSpec(memory_space=_VMEM),
            pl.BlockSpec(memory_space=_ANY),
        ],
        out_specs=pl.BlockSpec(memory_space=_VMEM),
        scratch_shapes=[
            pltpu.VMEM((2, D, D), jnp.float32),
            pltpu.SemaphoreType.DMA((2,)),
        ],
    )(Eoh, recv, W)


def _scatter_return(Et, Y):

    def body(et_ref, y_ref, ret_ref, yp_ref, send_s, recv_s, loc):
        my = lax.axis_index("i")
        _entry_barrier(my)
        _pipelined_pack_a2a(
            my, et_ref, y_ref, ret_ref, yp_ref, send_s, recv_s, loc
        )

    return pl.pallas_call(
        body,
        out_shape=jax.ShapeDtypeStruct((NP, D), jnp.bfloat16),
        in_specs=[
            pl.BlockSpec(memory_space=_VMEM),
            pl.BlockSpec(memory_space=_VMEM),
        ],
        out_specs=pl.BlockSpec(memory_space=_VMEM),
        scratch_shapes=[
            pltpu.VMEM((NP, D), jnp.bfloat16),
            pltpu.SemaphoreType.DMA((N_DEV - 1,)),
            pltpu.SemaphoreType.DMA((N_DEV - 1,)),
            pltpu.SemaphoreType.DMA,
        ],
        compiler_params=_CompilerParams(collective_id=2),
    )(Et, Y)


def _unpack(Pt, ret):

    def body(pt_ref, ret_ref, o_ref):
        o_ref[...] = jnp.dot(
            pt_ref[...], ret_ref[...], preferred_element_type=jnp.float32
        )

    return pl.pallas_call(
        body,
        out_shape=jax.ShapeDtypeStruct((N_LOC, D), jnp.float32),
        in_specs=[
            pl.BlockSpec(memory_space=_VMEM),
            pl.BlockSpec(memory_space=_VMEM),
        ],
        out_specs=pl.BlockSpec(memory_space=_VMEM),
    )(Pt, ret)


def kernel(x, router_W, route_idx, expert_W):
    del router_W
    my = lax.axis_index("i")

    R_all = _route_allgather(route_idx.reshape(16, 128))
    R = R_all.reshape(NT)

    onehot = (R[:, None] == jnp.arange(E, dtype=jnp.int32)[None, :]).astype(
        jnp.int32
    )
    inc = jnp.cumsum(onehot, axis=0)
    pos = jnp.sum(inc * onehot, axis=1) - 1
    accept = pos < CAP
    dest = R // E_LOC

    d_loc = lax.dynamic_slice_in_dim(dest, my * N_LOC, N_LOC)
    a_loc = lax.dynamic_slice_in_dim(accept.astype(jnp.int32), my * N_LOC, N_LOC)
    p_loc = lax.dynamic_slice_in_dim(pos, my * N_LOC, N_LOC)
    le_loc = lax.dynamic_slice_in_dim(R % E_LOC, my * N_LOC, N_LOC)
    oh8 = (d_loc[:, None] == jnp.arange(N_DEV, dtype=jnp.int32)[None, :]).astype(
        jnp.int32
    ) * a_loc[:, None]
    slot_loc = jnp.sum((jnp.cumsum(oh8, axis=0) - oh8) * oh8, axis=1)
    valid_loc = (a_loc == 1) & (slot_loc < PAD)
    key_loc = jnp.where(valid_loc, d_loc * PAD + slot_loc, BIG)
    rows = jnp.arange(NP, dtype=jnp.int32)
    P = (rows[:, None] == key_loc[None, :]).astype(jnp.bfloat16)
    Pt = (key_loc[:, None] == rows[None, :]).astype(jnp.bfloat16)

    lane = jnp.arange(128, dtype=jnp.int32)[None, :]
    augf = jnp.where(
        lane == 0,
        le_loc[:, None].astype(jnp.float32),
        jnp.where(
            lane == 1,
            p_loc[:, None].astype(jnp.float32),
            jnp.where(lane == 2, 1.0, 0.0),
        ),
    )
    x_aug = jnp.concatenate(
        [x.astype(jnp.bfloat16), augf.astype(jnp.bfloat16)], axis=1
    )

    recv = _pack_dispatch(P, x_aug)

    le_r = recv[:, D].astype(jnp.int32)
    pos_r = recv[:, D + 1].astype(jnp.int32)
    valid_r = recv[:, D + 2] > 0.5
    col_key = jnp.where(valid_r, le_r * ECAP + pos_r, BIG)
    erows = jnp.arange(E_LOC * ECAP, dtype=jnp.int32)
    Eoh = (erows[:, None] == col_key[None, :]).astype(jnp.bfloat16)
    Et = (col_key[:, None] == erows[None, :]).astype(jnp.bfloat16)

    Y = _expert_compute(Eoh, recv, expert_W)
    ret = _scatter_return(Et, Y)
    out = _unpack(Pt, ret)
    return out


# device time: 178714 ns/iter; 5.3627x vs baseline; 1.0005x over previous
import jax
import jax.numpy as jnp
from jax import lax
from jax.experimental import pallas as pl
from jax.experimental.pallas import tpu as pltpu

N_DEV = 8
N_LOC = 2048
D = 1024
DA = D + 128
E = 64
E_LOC = E // N_DEV
CAP = 204
PAD = 320
ECAP = 256
NP = N_DEV * PAD
NT = N_DEV * N_LOC
BIG = jnp.int32(1_000_000)

_DeviceIdType = getattr(pl, "DeviceIdType", None) or getattr(pltpu, "DeviceIdType")
MESH = _DeviceIdType.MESH
_sem_signal = getattr(pl, "semaphore_signal", None) or getattr(pltpu, "semaphore_signal")
_sem_wait = getattr(pl, "semaphore_wait", None) or getattr(pltpu, "semaphore_wait")
_CompilerParams = getattr(pltpu, "CompilerParams", None) or getattr(
    pltpu, "TPUCompilerParams"
)
_ANY = getattr(pltpu, "ANY", None) or pl.ANY
_VMEM = getattr(pltpu, "VMEM", None) or pltpu.MemorySpace.VMEM


def _entry_barrier(my):
    bar = pltpu.get_barrier_semaphore()
    for off in range(1, N_DEV):
        _sem_signal(
            bar,
            inc=1,
            device_id=((my + off) % N_DEV,),
            device_id_type=MESH,
        )
    _sem_wait(bar, N_DEV - 1)


def _route_allgather(r2):

    def body(r_ref, rall_ref, send_s, recv_s, loc):
        my = lax.axis_index("i")
        _entry_barrier(my)
        cp = pltpu.make_async_copy(r_ref, rall_ref.at[my], loc)
        cp.start()
        sends = []
        for off in range(1, N_DEV):
            d = (my + off) % N_DEV
            j = off - 1
            s = pltpu.make_async_remote_copy(
                src_ref=r_ref,
                dst_ref=rall_ref.at[my],
                send_sem=send_s.at[j],
                recv_sem=recv_s.at[j],
                device_id=(d,),
                device_id_type=MESH,
            )
            s.start()
            sends.append(s)
        for offp in range(1, N_DEV):
            k = (my - offp) % N_DEV
            j = offp - 1
            r = pltpu.make_async_remote_copy(
                src_ref=r_ref,
                dst_ref=rall_ref.at[k],
                send_sem=send_s.at[j],
                recv_sem=recv_s.at[j],
                device_id=(k,),
                device_id_type=MESH,
            )
            r.wait_recv()
        for s in sends:
            s.wait_send()
        cp.wait()

    return pl.pallas_call(
        body,
        out_shape=jax.ShapeDtypeStruct((N_DEV, 16, 128), jnp.int32),
        in_specs=[pl.BlockSpec(memory_space=_VMEM)],
        out_specs=pl.BlockSpec(memory_space=_VMEM),
        scratch_shapes=[
            pltpu.SemaphoreType.DMA((N_DEV - 1,)),
            pltpu.SemaphoreType.DMA((N_DEV - 1,)),
            pltpu.SemaphoreType.DMA,
        ],
        compiler_params=_CompilerParams(collective_id=0),
    )(r2)


def _pipelined_pack_a2a(my, m_ref, rhs_ref, dst_ref, stage_ref, send_s, recv_s, loc):
    sends = []
    cp = None
    for off in range(N_DEV):
        d = (my + off) % N_DEV
        sec = jnp.dot(
            m_ref[pl.ds(d * PAD, PAD), :],
            rhs_ref[...],
            preferred_element_type=jnp.float32,
        ).astype(stage_ref.dtype)
        stage_ref[pl.ds(d * PAD, PAD), :] = sec
        if off == 0:
            cp = pltpu.make_async_copy(
                stage_ref.at[pl.ds(my * PAD, PAD), :],
                dst_ref.at[pl.ds(my * PAD, PAD), :],
                loc,
            )
            cp.start()
        else:
            j = off - 1
            s = pltpu.make_async_remote_copy(
                src_ref=stage_ref.at[pl.ds(d * PAD, PAD), :],
                dst_ref=dst_ref.at[pl.ds(my * PAD, PAD), :],
                send_sem=send_s.at[j],
                recv_sem=recv_s.at[j],
                device_id=(d,),
                device_id_type=MESH,
            )
            s.start()
            sends.append(s)
    for offp in range(1, N_DEV):
        k = (my - offp) % N_DEV
        j = offp - 1
        r = pltpu.make_async_remote_copy(
            src_ref=stage_ref.at[pl.ds(k * PAD, PAD), :],
            dst_ref=dst_ref.at[pl.ds(k * PAD, PAD), :],
            send_sem=send_s.at[j],
            recv_sem=recv_s.at[j],
            device_id=(k,),
            device_id_type=MESH,
        )
        r.wait_recv()
    for s in sends:
        s.wait_send()
    cp.wait()


def _pack_dispatch(P, x_aug):

    def body(p_ref, x_ref, recv_ref, packed_ref, send_s, recv_s, loc):
        my = lax.axis_index("i")
        _entry_barrier(my)
        _pipelined_pack_a2a(
            my, p_ref, x_ref, recv_ref, packed_ref, send_s, recv_s, loc
        )

    return pl.pallas_call(
        body,
        out_shape=jax.ShapeDtypeStruct((NP, DA), jnp.bfloat16),
        in_specs=[
            pl.BlockSpec(memory_space=_VMEM),
            pl.BlockSpec(memory_space=_VMEM),
        ],
        out_specs=pl.BlockSpec(memory_space=_VMEM),
        scratch_shapes=[
            pltpu.VMEM((NP, DA), jnp.bfloat16),
            pltpu.SemaphoreType.DMA((N_DEV - 1,)),
            pltpu.SemaphoreType.DMA((N_DEV - 1,)),
            pltpu.SemaphoreType.DMA,
        ],
        compiler_params=_CompilerParams(collective_id=1),
    )(P, x_aug)


def _expert_compute(Eoh, recv, W):

    def body(e_ref, r_ref, w_ref, y_ref, wbuf, wsems):
        cp0 = pltpu.make_async_copy(w_ref.at[0], wbuf.at[0], wsems.at[0])
        cp0.start()
        xg = jnp.dot(
            e_ref[...], r_ref[:, :D], preferred_element_type=jnp.float32
        ).astype(jnp.bfloat16)
        for e in range(E_LOC):
            pltpu.make_async_copy(
                w_ref.at[e], wbuf.at[e % 2], wsems.at[e % 2]
            ).wait()
            if e + 1 < E_LOC:
                pltpu.make_async_copy(
                    w_ref.at[e + 1], wbuf.at[(e + 1) % 2], wsems.at[(e + 1) % 2]
                ).start()
            y = jnp.dot(
                xg[e * ECAP : (e + 1) * ECAP, :],
                wbuf[e % 2].astype(jnp.bfloat16),
                preferred_element_type=jnp.float32,
            )
            y_ref[e * ECAP : (e + 1) * ECAP, :] = y.astype(jnp.bfloat16)

    return pl.pallas_call(
        body,
        out_shape=jax.ShapeDtypeStruct((E_LOC * ECAP, D), jnp.bfloat16),
        in_specs=[
            pl.BlockSpec(memory_space=_VMEM),
            pl.BlockSpec(memory_space=_VMEM),
            pl.BlockSpec(memory_space=_ANY),
        ],
        out_specs=pl.BlockSpec(memory_space=_VMEM),
        scratch_shapes=[
            pltpu.VMEM((2, D, D), jnp.float32),
            pltpu.SemaphoreType.DMA((2,)),
        ],
    )(Eoh, recv, W)


def _scatter_return(Et, Y):

    def body(et_ref, y_ref, ret_ref, yp_ref, send_s, recv_s, loc):
        my = lax.axis_index("i")
        _entry_barrier(my)
        _pipelined_pack_a2a(
            my, et_ref, y_ref, ret_ref, yp_ref, send_s, recv_s, loc
        )

    return pl.pallas_call(
        body,
        out_shape=jax.ShapeDtypeStruct((NP, D), jnp.bfloat16),
        in_specs=[
            pl.BlockSpec(memory_space=_VMEM),
            pl.BlockSpec(memory_space=_VMEM),
        ],
        out_specs=pl.BlockSpec(memory_space=_VMEM),
        scratch_shapes=[
            pltpu.VMEM((NP, D), jnp.bfloat16),
            pltpu.SemaphoreType.DMA((N_DEV - 1,)),
            pltpu.SemaphoreType.DMA((N_DEV - 1,)),
            pltpu.SemaphoreType.DMA,
        ],
        compiler_params=_CompilerParams(collective_id=2),
    )(Et, Y)


def _unpack(Pt, ret):

    def body(pt_ref, ret_ref, o_ref):
        o_ref[...] = jnp.dot(
            pt_ref[...], ret_ref[...], preferred_element_type=jnp.float32
        )

    return pl.pallas_call(
        body,
        out_shape=jax.ShapeDtypeStruct((N_LOC, D), jnp.float32),
        in_specs=[
            pl.BlockSpec(memory_space=_VMEM),
            pl.BlockSpec(memory_space=_VMEM),
        ],
        out_specs=pl.BlockSpec(memory_space=_VMEM),
    )(Pt, ret)


def kernel(x, router_W, route_idx, expert_W):
    del router_W
    my = lax.axis_index("i")

    R_all = _route_allgather(route_idx.reshape(16, 128))
    R = R_all.reshape(NT)

    onehot = (R[:, None] == jnp.arange(E, dtype=jnp.int32)[None, :]).astype(
        jnp.int32
    )
    inc = jnp.cumsum(onehot, axis=0)
    pos = jnp.sum(inc * onehot, axis=1) - 1
    accept = pos < CAP
    dest = R // E_LOC

    d_loc = lax.dynamic_slice_in_dim(dest, my * N_LOC, N_LOC)
    a_loc = lax.dynamic_slice_in_dim(accept.astype(jnp.int32), my * N_LOC, N_LOC)
    p_loc = lax.dynamic_slice_in_dim(pos, my * N_LOC, N_LOC)
    le_loc = lax.dynamic_slice_in_dim(R % E_LOC, my * N_LOC, N_LOC)
    oh8 = (d_loc[:, None] == jnp.arange(N_DEV, dtype=jnp.int32)[None, :]).astype(
        jnp.int32
    ) * a_loc[:, None]
    slot_loc = jnp.sum((jnp.cumsum(oh8, axis=0) - oh8) * oh8, axis=1)
    valid_loc = (a_loc == 1) & (slot_loc < PAD)
    key_loc = jnp.where(valid_loc, d_loc * PAD + slot_loc, BIG)
    rows = jnp.arange(NP, dtype=jnp.int32)
    P = (rows[:, None] == key_loc[None, :]).astype(jnp.bfloat16)
    Pt = (key_loc[:, None] == rows[None, :]).astype(jnp.bfloat16)

    lane = jnp.arange(128, dtype=jnp.int32)[None, :]
    augf = jnp.where(
        lane == 0,
        le_loc[:, None].astype(jnp.float32),
        jnp.where(
            lane == 1,
            p_loc[:, None].astype(jnp.float32),
            jnp.where(lane == 2, 1.0, 0.0),
        ),
    )
    x_aug = jnp.concatenate(
        [x.astype(jnp.bfloat16), augf.astype(jnp.bfloat16)], axis=1
    )

    recv = _pack_dispatch(P, x_aug)

    le_r = recv[:, D].astype(jnp.int32)
    pos_r = recv[:, D + 1].astype(jnp.int32)
    valid_r = recv[:, D + 2] > 0.5
    col_key = jnp.where(valid_r, le_r * ECAP + pos_r, BIG)
    erows = jnp.arange(E_LOC * ECAP, dtype=jnp.int32)
    Eoh = (erows[:, None] == col_key[None, :]).astype(jnp.bfloat16)
    Et = (col_key[:, None] == erows[None, :]).astype(jnp.bfloat16)

    Y = _expert_compute(Eoh, recv, expert_W)
    ret = _scatter_return(Et, Y)
    out = _unpack(Pt, ret)
    return out
